# baseline (speedup 1.0000x reference)
import functools

import jax
import jax.numpy as jnp
from jax import lax
from jax.experimental import pallas as pl
from jax.experimental.pallas import tpu as pltpu
from jax.experimental.pallas import tpu_sc as plsc

_K_BOND = 20.0
_NC = 2
_NS = 16
_NW = _NC * _NS
_C = 128


def _rsqrt(s):
    i = lax.bitcast_convert_type(s, jnp.int32)
    i = jnp.int32(0x5F3759DF) - lax.shift_right_logical(i, jnp.int32(1))
    y = lax.bitcast_convert_type(i, jnp.float32)
    for _ in range(3):
        y = y * (1.5 - 0.5 * s * y * y)
    return y


def _make_sc_kernel(n_chunks, G, S, rps):
    mesh = plsc.VectorSubcoreMesh(core_axis_name="c", subcore_axis_name="s")
    npc = n_chunks * _C

    scratch = (
        [pltpu.VMEM((n_chunks, _C), jnp.int32)] * 3
        + [pltpu.VMEM((npc,), jnp.float32)]
        + [pltpu.VMEM((_C,), jnp.float32)] * 24
        + [pltpu.VMEM((_C,), jnp.float32)] * 28
        + [pltpu.VMEM((_C,), jnp.int32)] * 24
        + [pltpu.VMEM((G * 4 // _NS,), jnp.float32)]
        + [pltpu.VMEM_SHARED((G * 4,), jnp.float32)]
        + [pltpu.VMEM_SHARED((G,), jnp.float32)] * 3
        + [pltpu.VMEM_SHARED((S,), jnp.float32)]
        + [pltpu.SemaphoreType.DMA] * 8
    )

    @functools.partial(
        pl.kernel,
        out_type=[
            jax.ShapeDtypeStruct((_NC * 3 * G,), jnp.float32),
            jax.ShapeDtypeStruct((_NC * S,), jnp.float32),
        ],
        mesh=mesh,
        scratch_types=scratch,
    )
    def sc_kernel(tbl, srcb, dstb, segb, r0b, zeros, gpart, epart,
                  *refs):
        srcv, dstv, segv, r0v = refs[0:4]
        o = 4
        gbuf = tuple(refs[o + 6 * b:o + 6 * (b + 1)] for b in range(4))
        o += 24
        vbuf = tuple(refs[o + 7 * b:o + 7 * (b + 1)] for b in range(4))
        o += 28
        ibuf = tuple(refs[o + 6 * b:o + 6 * (b + 1)] for b in range(4))
        o += 24
        stage = refs[o]
        tb_sh = refs[o + 1]
        gxa, gya, gza, e_sh = refs[o + 2:o + 6]
        semg = refs[o + 6:o + 10]
        sems = refs[o + 10:o + 14]

        c = lax.axis_index("c")
        s = lax.axis_index("s")
        wid = s * _NC + c

        pltpu.sync_copy(srcb.at[wid], srcv)
        pltpu.sync_copy(dstb.at[wid], dstv)
        pltpu.sync_copy(segb.at[wid], segv)
        pltpu.sync_copy(r0b.at[pl.ds(wid * npc, npc)], r0v)

        tps = G * 4 // _NS
        pltpu.sync_copy(tbl.at[pl.ds(s * tps, tps)], stage)
        pltpu.sync_copy(stage, tb_sh.at[pl.ds(s * tps, tps)])

        pltpu.sync_copy(zeros, stage.at[pl.ds(0, rps)])
        pltpu.sync_copy(stage.at[pl.ds(0, rps)], gxa.at[pl.ds(s * rps, rps)])
        pltpu.sync_copy(stage.at[pl.ds(0, rps)], gya.at[pl.ds(s * rps, rps)])
        pltpu.sync_copy(stage.at[pl.ds(0, rps)], gza.at[pl.ds(s * rps, rps)])

        @pl.when(s == 0)
        def _():
            pltpu.sync_copy(stage.at[pl.ds(0, S)], e_sh)

        plsc.subcore_barrier()

        def build_idx(j, b):
            for k in range(_C // 16):
                sl = pl.ds(k * 16, 16)
                s4 = lax.shift_left(srcv[j, sl], jnp.int32(2))
                d4 = lax.shift_left(dstv[j, sl], jnp.int32(2))
                ibuf[b][0][sl] = s4 + 1
                ibuf[b][1][sl] = s4 + 2
                ibuf[b][2][sl] = s4 + 3
                ibuf[b][3][sl] = d4 + 1
                ibuf[b][4][sl] = d4 + 2
                ibuf[b][5][sl] = d4 + 3

        def issue_gathers(b):
            for i in range(6):
                pltpu.async_copy(tb_sh.at[ibuf[b][i]], gbuf[b][i], semg[b])

        def wait_gathers(b):
            for dst in gbuf[b]:
                pltpu.make_async_copy(tb_sh.at[ibuf[b][0]], dst,
                                      semg[b]).wait()

        def issue_scatters(j, b):
            gx, gy, gz, nx, ny, nz, ev = vbuf[b]
            pltpu.async_copy(gx, gxa.at[srcv.at[j]], sems[b], add=True)
            pltpu.async_copy(gy, gya.at[srcv.at[j]], sems[b], add=True)
            pltpu.async_copy(gz, gza.at[srcv.at[j]], sems[b], add=True)
            pltpu.async_copy(nx, gxa.at[dstv.at[j]], sems[b], add=True)
            pltpu.async_copy(ny, gya.at[dstv.at[j]], sems[b], add=True)
            pltpu.async_copy(nz, gza.at[dstv.at[j]], sems[b], add=True)
            pltpu.async_copy(ev, e_sh.at[segv.at[j]], sems[b], add=True)

        def wait_scatters(b):
            for src in vbuf[b]:
                pltpu.make_async_copy(src, gxa.at[srcv.at[0]], sems[b]).wait()

        def compute(j, b):
            sxv, syv, szv, txv, tyv, tzv = gbuf[b]
            gxv, gyv, gzv, nxv, nyv, nzv, ev = vbuf[b]
            for k in range(_C // 16):
                sl = pl.ds(k * 16, 16)
                dx = sxv[sl] - txv[sl]
                dy = syv[sl] - tyv[sl]
                dz = szv[sl] - tzv[sl]
                ssq = dx * dx + dy * dy + dz * dz
                y = _rsqrt(ssq)
                r0_ = r0v[pl.ds(j * _C + k * 16, 16)]
                diff = ssq * y - r0_
                e = _K_BOND * diff * diff
                coef = (2.0 * _K_BOND) * diff * y
                gx = coef * dx
                gy = coef * dy
                gz = coef * dz
                gxv[sl] = gx
                gyv[sl] = gy
                gzv[sl] = gz
                nxv[sl] = -gx
                nyv[sl] = -gy
                nzv[sl] = -gz
                ev[sl] = e

        build_idx(0, 0)
        issue_gathers(0)
        build_idx(1, 1)
        issue_gathers(1)

        def quad(jj, carry):
            for b in range(4):
                jc = 4 * jj + b
                nb = (b + 2) % 4

                @pl.when(jc + 2 < n_chunks)
                def _():
                    build_idx(jc + 2, nb)
                    issue_gathers(nb)

                @pl.when(jc >= 4)
                def _():
                    wait_scatters(b)

                wait_gathers(b)
                compute(jc, b)
                issue_scatters(jc, b)
            return carry

        lax.fori_loop(0, n_chunks // 4, quad, 0)
        for b in range(4):
            wait_scatters(b)
        plsc.subcore_barrier()

        st = stage.at[pl.ds(0, rps)]
        pltpu.sync_copy(gxa.at[pl.ds(s * rps, rps)], st)
        pltpu.sync_copy(st, gpart.at[pl.ds((c * 3 + 0) * G + s * rps, rps)])
        pltpu.sync_copy(gya.at[pl.ds(s * rps, rps)], st)
        pltpu.sync_copy(st, gpart.at[pl.ds((c * 3 + 1) * G + s * rps, rps)])
        pltpu.sync_copy(gza.at[pl.ds(s * rps, rps)], st)
        pltpu.sync_copy(st, gpart.at[pl.ds((c * 3 + 2) * G + s * rps, rps)])

        @pl.when(s == 0)
        def _():
            pltpu.sync_copy(e_sh, stage.at[pl.ds(0, S)])
            pltpu.sync_copy(stage.at[pl.ds(0, S)], epart.at[pl.ds(c * S, S)])

    return sc_kernel


def _combine_body(g_ref, e_ref, go_ref, eo_ref):
    go_ref[...] = g_ref[0, :] + g_ref[1, :]
    eo_ref[...] = e_ref[0, :] + e_ref[1, :]


def kernel(nxyz, bonds, bond_len, num_bonds):
    n_atoms = nxyz.shape[0]
    n_bonds = bonds.shape[0]
    n_mol = num_bonds.shape[0]

    G = ((n_atoms + 2 + 127) // 128) * 128
    S = ((n_mol + 1 + 15) // 16) * 16
    rps = G // _NS

    chunks_total = -(-n_bonds // _C)
    n_chunks = -(-chunks_total // _NW)
    n_chunks = ((n_chunks + 3) // 4) * 4
    n_pad = n_chunks * _NW * _C

    tbl = jnp.concatenate(
        [nxyz.reshape(-1), jnp.zeros(((G - n_atoms) * 4,), jnp.float32)])

    pad = n_pad - n_bonds
    src = jnp.concatenate(
        [bonds[:, 0], jnp.full((pad,), n_atoms, jnp.int32)])
    dst = jnp.concatenate(
        [bonds[:, 1], jnp.full((pad,), n_atoms + 1, jnp.int32)])
    starts = jnp.cumsum(num_bonds)[:-1]
    mark = jnp.zeros((n_bonds,), jnp.int32).at[starts].add(1)
    seg = jnp.concatenate([
        jnp.cumsum(mark, dtype=jnp.int32),
        jnp.full((pad,), n_mol, jnp.int32),
    ])
    r0 = jnp.concatenate([bond_len[:, 0], jnp.zeros((pad,), jnp.float32)])
    zeros = jnp.zeros((rps,), jnp.float32)

    sc_kernel = _make_sc_kernel(n_chunks, G, S, rps)
    gpart, epart = sc_kernel(
        tbl,
        src.reshape(_NW, -1, _C), dst.reshape(_NW, -1, _C),
        seg.reshape(_NW, -1, _C), r0, zeros)

    gsum, esum = pl.pallas_call(
        _combine_body,
        out_shape=[
            jax.ShapeDtypeStruct((3 * G,), jnp.float32),
            jax.ShapeDtypeStruct((S,), jnp.float32),
        ],
    )(gpart.reshape(_NC, 3 * G), epart.reshape(_NC, S))

    g3 = gsum.reshape(3, G)
    energy_grad = jnp.stack(
        [g3[0, :n_atoms], g3[1, :n_atoms], g3[2, :n_atoms]], axis=1)
    E = esum[:n_mol].reshape(n_mol, 1)
    return E, energy_grad


# speedup vs baseline: 1.1820x; 1.1820x over previous
import functools

import jax
import jax.numpy as jnp
from jax import lax
from jax.experimental import pallas as pl
from jax.experimental.pallas import tpu as pltpu
from jax.experimental.pallas import tpu_sc as plsc

_K_BOND = 20.0
_NC = 2
_NS = 16
_NW = _NC * _NS
_C = 128


def _rsqrt(s):
    i = lax.bitcast_convert_type(s, jnp.int32)
    i = jnp.int32(0x5F3759DF) - lax.shift_right_logical(i, jnp.int32(1))
    y = lax.bitcast_convert_type(i, jnp.float32)
    for _ in range(2):
        y = y * (1.5 - 0.5 * s * y * y)
    return y


def _make_sc_kernel(n_chunks, G, S, rps):
    mesh = plsc.VectorSubcoreMesh(core_axis_name="c", subcore_axis_name="s")
    npc = n_chunks * _C

    scratch = (
        [pltpu.VMEM((n_chunks, _C), jnp.int32)] * 3
        + [pltpu.VMEM((npc,), jnp.float32)]
        + [pltpu.VMEM((_C,), jnp.float32)] * 12
        + [pltpu.VMEM((_C,), jnp.float32)] * 14
        + [pltpu.VMEM((_C,), jnp.int32)] * 12
        + [pltpu.VMEM((G * 4 // _NS,), jnp.float32)]
        + [pltpu.VMEM_SHARED((G * 4,), jnp.float32)]
        + [pltpu.VMEM_SHARED((G,), jnp.float32)] * 3
        + [pltpu.VMEM_SHARED((S,), jnp.float32)]
        + [pltpu.SemaphoreType.DMA] * 4
    )

    @functools.partial(
        pl.kernel,
        out_type=[
            jax.ShapeDtypeStruct((_NC * 3 * G,), jnp.float32),
            jax.ShapeDtypeStruct((_NC * S,), jnp.float32),
        ],
        mesh=mesh,
        scratch_types=scratch,
    )
    def sc_kernel(tbl, srcb, dstb, segb, r0b, zeros, gpart, epart,
                  *refs):
        srcv, dstv, segv, r0v = refs[0:4]
        o = 4
        gbuf = tuple(refs[o + 6 * b:o + 6 * (b + 1)] for b in range(2))
        o += 12
        vbuf = tuple(refs[o + 7 * b:o + 7 * (b + 1)] for b in range(2))
        o += 14
        ibuf = tuple(refs[o + 6 * b:o + 6 * (b + 1)] for b in range(2))
        o += 12
        stage = refs[o]
        tb_sh = refs[o + 1]
        gxa, gya, gza, e_sh = refs[o + 2:o + 6]
        semg = refs[o + 6:o + 8]
        sems = refs[o + 8:o + 10]

        c = lax.axis_index("c")
        s = lax.axis_index("s")
        wid = s * _NC + c

        tps = G * 4 // _NS
        pltpu.async_copy(srcb.at[wid], srcv, semg[0])
        pltpu.async_copy(dstb.at[wid], dstv, semg[0])
        pltpu.async_copy(segb.at[wid], segv, semg[0])
        pltpu.async_copy(r0b.at[pl.ds(wid * npc, npc)], r0v, semg[0])
        pltpu.async_copy(tbl.at[pl.ds(s * tps, tps)], stage, semg[0])
        pltpu.make_async_copy(srcb.at[wid], srcv, semg[0]).wait()
        pltpu.make_async_copy(dstb.at[wid], dstv, semg[0]).wait()
        pltpu.make_async_copy(segb.at[wid], segv, semg[0]).wait()
        pltpu.make_async_copy(r0b.at[pl.ds(wid * npc, npc)], r0v,
                              semg[0]).wait()
        pltpu.make_async_copy(tbl.at[pl.ds(s * tps, tps)], stage,
                              semg[0]).wait()

        pltpu.sync_copy(stage, tb_sh.at[pl.ds(s * tps, tps)])
        pltpu.sync_copy(zeros, stage.at[pl.ds(0, rps)])
        pltpu.async_copy(stage.at[pl.ds(0, rps)],
                         gxa.at[pl.ds(s * rps, rps)], semg[0])
        pltpu.async_copy(stage.at[pl.ds(0, rps)],
                         gya.at[pl.ds(s * rps, rps)], semg[0])
        pltpu.async_copy(stage.at[pl.ds(0, rps)],
                         gza.at[pl.ds(s * rps, rps)], semg[0])
        pltpu.make_async_copy(stage.at[pl.ds(0, rps)],
                              gxa.at[pl.ds(s * rps, rps)], semg[0]).wait()
        pltpu.make_async_copy(stage.at[pl.ds(0, rps)],
                              gya.at[pl.ds(s * rps, rps)], semg[0]).wait()
        pltpu.make_async_copy(stage.at[pl.ds(0, rps)],
                              gza.at[pl.ds(s * rps, rps)], semg[0]).wait()

        @pl.when(s == 0)
        def _():
            pltpu.sync_copy(stage.at[pl.ds(0, S)], e_sh)

        plsc.subcore_barrier()

        def build_idx(j, b):
            for k in range(_C // 16):
                sl = pl.ds(k * 16, 16)
                s4 = lax.shift_left(srcv[j, sl], jnp.int32(2))
                d4 = lax.shift_left(dstv[j, sl], jnp.int32(2))
                ibuf[b][0][sl] = s4 + 1
                ibuf[b][1][sl] = s4 + 2
                ibuf[b][2][sl] = s4 + 3
                ibuf[b][3][sl] = d4 + 1
                ibuf[b][4][sl] = d4 + 2
                ibuf[b][5][sl] = d4 + 3

        def issue_gathers(b):
            for i in range(6):
                pltpu.async_copy(tb_sh.at[ibuf[b][i]], gbuf[b][i], semg[b])

        def wait_gathers(b):
            for dst in gbuf[b]:
                pltpu.make_async_copy(tb_sh.at[ibuf[b][0]], dst,
                                      semg[b]).wait()

        def issue_scatters(j, b):
            gx, gy, gz, nx, ny, nz, ev = vbuf[b]
            pltpu.async_copy(gx, gxa.at[srcv.at[j]], sems[b], add=True)
            pltpu.async_copy(gy, gya.at[srcv.at[j]], sems[b], add=True)
            pltpu.async_copy(gz, gza.at[srcv.at[j]], sems[b], add=True)
            pltpu.async_copy(nx, gxa.at[dstv.at[j]], sems[b], add=True)
            pltpu.async_copy(ny, gya.at[dstv.at[j]], sems[b], add=True)
            pltpu.async_copy(nz, gza.at[dstv.at[j]], sems[b], add=True)
            pltpu.async_copy(ev, e_sh.at[segv.at[j]], sems[b], add=True)

        def wait_scatters(b):
            for src in vbuf[b]:
                pltpu.make_async_copy(src, gxa.at[srcv.at[0]], sems[b]).wait()

        def compute(j, b):
            sxv, syv, szv, txv, tyv, tzv = gbuf[b]
            gxv, gyv, gzv, nxv, nyv, nzv, ev = vbuf[b]
            for k in range(_C // 16):
                sl = pl.ds(k * 16, 16)
                dx = sxv[sl] - txv[sl]
                dy = syv[sl] - tyv[sl]
                dz = szv[sl] - tzv[sl]
                ssq = dx * dx + dy * dy + dz * dz
                y = _rsqrt(ssq)
                r0_ = r0v[pl.ds(j * _C + k * 16, 16)]
                diff = ssq * y - r0_
                e = _K_BOND * diff * diff
                coef = (2.0 * _K_BOND) * diff * y
                gx = coef * dx
                gy = coef * dy
                gz = coef * dz
                gxv[sl] = gx
                gyv[sl] = gy
                gzv[sl] = gz
                nxv[sl] = -gx
                nyv[sl] = -gy
                nzv[sl] = -gz
                ev[sl] = e

        build_idx(0, 0)
        issue_gathers(0)

        def pair(jj, carry):
            for b in range(2):
                jc = 2 * jj + b
                nb = 1 - b

                @pl.when(jc + 1 < n_chunks)
                def _():
                    build_idx(jc + 1, nb)
                    issue_gathers(nb)

                @pl.when(jc >= 2)
                def _():
                    wait_scatters(b)

                wait_gathers(b)
                compute(jc, b)
                issue_scatters(jc, b)
            return carry

        lax.fori_loop(0, n_chunks // 2, pair, 0)
        wait_scatters(0)
        wait_scatters(1)
        plsc.subcore_barrier()

        st = stage.at[pl.ds(0, rps)]
        pltpu.sync_copy(gxa.at[pl.ds(s * rps, rps)], st)
        pltpu.sync_copy(st, gpart.at[pl.ds((c * 3 + 0) * G + s * rps, rps)])
        pltpu.sync_copy(gya.at[pl.ds(s * rps, rps)], st)
        pltpu.sync_copy(st, gpart.at[pl.ds((c * 3 + 1) * G + s * rps, rps)])
        pltpu.sync_copy(gza.at[pl.ds(s * rps, rps)], st)
        pltpu.sync_copy(st, gpart.at[pl.ds((c * 3 + 2) * G + s * rps, rps)])

        @pl.when(s == 0)
        def _():
            pltpu.sync_copy(e_sh, stage.at[pl.ds(0, S)])
            pltpu.sync_copy(stage.at[pl.ds(0, S)], epart.at[pl.ds(c * S, S)])

    return sc_kernel


def _combine_body(g_ref, e_ref, go_ref, eo_ref):
    go_ref[...] = g_ref[0, :] + g_ref[1, :]
    eo_ref[...] = e_ref[0, :] + e_ref[1, :]


def kernel(nxyz, bonds, bond_len, num_bonds):
    n_atoms = nxyz.shape[0]
    n_bonds = bonds.shape[0]
    n_mol = num_bonds.shape[0]

    G = ((n_atoms + 2 + 127) // 128) * 128
    S = ((n_mol + 1 + 15) // 16) * 16
    rps = G // _NS

    chunks_total = -(-n_bonds // _C)
    n_chunks = -(-chunks_total // _NW)
    n_chunks += n_chunks % 2
    n_pad = n_chunks * _NW * _C

    tbl = jnp.concatenate(
        [nxyz.reshape(-1), jnp.zeros(((G - n_atoms) * 4,), jnp.float32)])

    pad = n_pad - n_bonds
    src = jnp.concatenate(
        [bonds[:, 0], jnp.full((pad,), n_atoms, jnp.int32)])
    dst = jnp.concatenate(
        [bonds[:, 1], jnp.full((pad,), n_atoms + 1, jnp.int32)])
    starts = jnp.cumsum(num_bonds)[:-1]
    mark = jnp.zeros((n_bonds,), jnp.int32).at[starts].add(1)
    seg = jnp.concatenate([
        jnp.cumsum(mark, dtype=jnp.int32),
        jnp.full((pad,), n_mol, jnp.int32),
    ])
    r0 = jnp.concatenate([bond_len[:, 0], jnp.zeros((pad,), jnp.float32)])
    zeros = jnp.zeros((rps,), jnp.float32)

    sc_kernel = _make_sc_kernel(n_chunks, G, S, rps)
    gpart, epart = sc_kernel(
        tbl,
        src.reshape(_NW, -1, _C), dst.reshape(_NW, -1, _C),
        seg.reshape(_NW, -1, _C), r0, zeros)

    gsum, esum = pl.pallas_call(
        _combine_body,
        out_shape=[
            jax.ShapeDtypeStruct((3 * G,), jnp.float32),
            jax.ShapeDtypeStruct((S,), jnp.float32),
        ],
    )(gpart.reshape(_NC, 3 * G), epart.reshape(_NC, S))

    g3 = gsum.reshape(3, G)
    energy_grad = jnp.stack(
        [g3[0, :n_atoms], g3[1, :n_atoms], g3[2, :n_atoms]], axis=1)
    E = esum[:n_mol].reshape(n_mol, 1)
    return E, energy_grad


# speedup vs baseline: 1.1833x; 1.0011x over previous
import functools

import jax
import jax.numpy as jnp
from jax import lax
from jax.experimental import pallas as pl
from jax.experimental.pallas import tpu as pltpu
from jax.experimental.pallas import tpu_sc as plsc

_K_BOND = 20.0
_NC = 2
_NS = 16
_NW = _NC * _NS
_C = 128


def _rsqrt(s):
    i = lax.bitcast_convert_type(s, jnp.int32)
    i = jnp.int32(0x5F3759DF) - lax.shift_right_logical(i, jnp.int32(1))
    y = lax.bitcast_convert_type(i, jnp.float32)
    for _ in range(2):
        y = y * (1.5 - 0.5 * s * y * y)
    return y


def _make_sc_kernel(n_chunks, G, S, rps):
    mesh = plsc.VectorSubcoreMesh(core_axis_name="c", subcore_axis_name="s")
    npc = n_chunks * _C

    scratch = (
        [pltpu.VMEM((n_chunks, _C), jnp.int32)] * 3
        + [pltpu.VMEM((npc,), jnp.float32)]
        + [pltpu.VMEM((_C,), jnp.float32)] * 12
        + [pltpu.VMEM((_C,), jnp.float32)] * 14
        + [pltpu.VMEM((_C,), jnp.int32)] * 12
        + [pltpu.VMEM((G * 4 // _NS,), jnp.float32)]
        + [pltpu.VMEM_SHARED((G * 4,), jnp.float32)]
        + [pltpu.VMEM_SHARED((G,), jnp.float32)] * 3
        + [pltpu.VMEM_SHARED((S,), jnp.float32)]
        + [pltpu.SemaphoreType.DMA] * 4
    )

    @functools.partial(
        pl.kernel,
        out_type=[
            jax.ShapeDtypeStruct((_NC * 3 * G,), jnp.float32),
            jax.ShapeDtypeStruct((_NC * S,), jnp.float32),
        ],
        mesh=mesh,
        scratch_types=scratch,
    )
    def sc_kernel(tbl, srcb, dstb, segb, r0b, zeros, gpart, epart,
                  *refs):
        srcv, dstv, segv, r0v = refs[0:4]
        o = 4
        gbuf = tuple(refs[o + 6 * b:o + 6 * (b + 1)] for b in range(2))
        o += 12
        vbuf = tuple(refs[o + 7 * b:o + 7 * (b + 1)] for b in range(2))
        o += 14
        ibuf = tuple(refs[o + 6 * b:o + 6 * (b + 1)] for b in range(2))
        o += 12
        stage = refs[o]
        tb_sh = refs[o + 1]
        gxa, gya, gza, e_sh = refs[o + 2:o + 6]
        semg = refs[o + 6:o + 8]
        sems = refs[o + 8:o + 10]

        c = lax.axis_index("c")
        s = lax.axis_index("s")
        wid = s * _NC + c

        tps = G * 4 // _NS
        pltpu.async_copy(srcb.at[wid], srcv, semg[0])
        pltpu.async_copy(dstb.at[wid], dstv, semg[0])
        pltpu.async_copy(segb.at[wid], segv, semg[0])
        pltpu.async_copy(r0b.at[pl.ds(wid * npc, npc)], r0v, semg[0])
        pltpu.async_copy(tbl.at[pl.ds(s * tps, tps)], stage, semg[0])
        pltpu.make_async_copy(srcb.at[wid], srcv, semg[0]).wait()
        pltpu.make_async_copy(dstb.at[wid], dstv, semg[0]).wait()
        pltpu.make_async_copy(segb.at[wid], segv, semg[0]).wait()
        pltpu.make_async_copy(r0b.at[pl.ds(wid * npc, npc)], r0v,
                              semg[0]).wait()
        pltpu.make_async_copy(tbl.at[pl.ds(s * tps, tps)], stage,
                              semg[0]).wait()

        pltpu.sync_copy(stage, tb_sh.at[pl.ds(s * tps, tps)])
        pltpu.sync_copy(zeros, stage.at[pl.ds(0, rps)])
        pltpu.async_copy(stage.at[pl.ds(0, rps)],
                         gxa.at[pl.ds(s * rps, rps)], semg[0])
        pltpu.async_copy(stage.at[pl.ds(0, rps)],
                         gya.at[pl.ds(s * rps, rps)], semg[0])
        pltpu.async_copy(stage.at[pl.ds(0, rps)],
                         gza.at[pl.ds(s * rps, rps)], semg[0])
        pltpu.make_async_copy(stage.at[pl.ds(0, rps)],
                              gxa.at[pl.ds(s * rps, rps)], semg[0]).wait()
        pltpu.make_async_copy(stage.at[pl.ds(0, rps)],
                              gya.at[pl.ds(s * rps, rps)], semg[0]).wait()
        pltpu.make_async_copy(stage.at[pl.ds(0, rps)],
                              gza.at[pl.ds(s * rps, rps)], semg[0]).wait()

        @pl.when(s == 0)
        def _():
            pltpu.sync_copy(stage.at[pl.ds(0, S)], e_sh)

        plsc.subcore_barrier()

        def build_idx(j, b):
            for k in range(_C // 16):
                sl = pl.ds(k * 16, 16)
                s4 = lax.shift_left(srcv[j, sl], jnp.int32(2))
                d4 = lax.shift_left(dstv[j, sl], jnp.int32(2))
                ibuf[b][0][sl] = s4 + 1
                ibuf[b][1][sl] = s4 + 2
                ibuf[b][2][sl] = s4 + 3
                ibuf[b][3][sl] = d4 + 1
                ibuf[b][4][sl] = d4 + 2
                ibuf[b][5][sl] = d4 + 3

        def issue_gathers(b):
            for i in range(6):
                pltpu.async_copy(tb_sh.at[ibuf[b][i]], gbuf[b][i], semg[b])

        def wait_gathers(b):
            pltpu.make_async_copy(zeros.at[pl.ds(0, 6 * _C)],
                                  stage.at[pl.ds(0, 6 * _C)],
                                  semg[b]).wait()

        def issue_scatters(j, b):
            gx, gy, gz, nx, ny, nz, ev = vbuf[b]
            pltpu.async_copy(gx, gxa.at[srcv.at[j]], sems[b], add=True)
            pltpu.async_copy(gy, gya.at[srcv.at[j]], sems[b], add=True)
            pltpu.async_copy(gz, gza.at[srcv.at[j]], sems[b], add=True)
            pltpu.async_copy(nx, gxa.at[dstv.at[j]], sems[b], add=True)
            pltpu.async_copy(ny, gya.at[dstv.at[j]], sems[b], add=True)
            pltpu.async_copy(nz, gza.at[dstv.at[j]], sems[b], add=True)
            pltpu.async_copy(ev, e_sh.at[segv.at[j]], sems[b], add=True)

        def wait_scatters(b):
            pltpu.make_async_copy(zeros.at[pl.ds(0, 7 * _C)],
                                  stage.at[pl.ds(0, 7 * _C)],
                                  sems[b]).wait()

        def compute(j, b):
            sxv, syv, szv, txv, tyv, tzv = gbuf[b]
            gxv, gyv, gzv, nxv, nyv, nzv, ev = vbuf[b]
            for k in range(_C // 16):
                sl = pl.ds(k * 16, 16)
                dx = sxv[sl] - txv[sl]
                dy = syv[sl] - tyv[sl]
                dz = szv[sl] - tzv[sl]
                ssq = dx * dx + dy * dy + dz * dz
                y = _rsqrt(ssq)
                r0_ = r0v[pl.ds(j * _C + k * 16, 16)]
                diff = ssq * y - r0_
                e = _K_BOND * diff * diff
                coef = (2.0 * _K_BOND) * diff * y
                gx = coef * dx
                gy = coef * dy
                gz = coef * dz
                gxv[sl] = gx
                gyv[sl] = gy
                gzv[sl] = gz
                nxv[sl] = -gx
                nyv[sl] = -gy
                nzv[sl] = -gz
                ev[sl] = e

        build_idx(0, 0)
        issue_gathers(0)

        def pair(jj, carry):
            for b in range(2):
                jc = 2 * jj + b
                nb = 1 - b

                @pl.when(jc + 1 < n_chunks)
                def _():
                    build_idx(jc + 1, nb)
                    issue_gathers(nb)

                @pl.when(jc >= 2)
                def _():
                    wait_scatters(b)

                wait_gathers(b)
                compute(jc, b)
                issue_scatters(jc, b)
            return carry

        lax.fori_loop(0, n_chunks // 2, pair, 0)
        wait_scatters(0)
        wait_scatters(1)
        plsc.subcore_barrier()

        st = stage.at[pl.ds(0, rps)]
        pltpu.sync_copy(gxa.at[pl.ds(s * rps, rps)], st)
        pltpu.sync_copy(st, gpart.at[pl.ds((c * 3 + 0) * G + s * rps, rps)])
        pltpu.sync_copy(gya.at[pl.ds(s * rps, rps)], st)
        pltpu.sync_copy(st, gpart.at[pl.ds((c * 3 + 1) * G + s * rps, rps)])
        pltpu.sync_copy(gza.at[pl.ds(s * rps, rps)], st)
        pltpu.sync_copy(st, gpart.at[pl.ds((c * 3 + 2) * G + s * rps, rps)])

        @pl.when(s == 0)
        def _():
            pltpu.sync_copy(e_sh, stage.at[pl.ds(0, S)])
            pltpu.sync_copy(stage.at[pl.ds(0, S)], epart.at[pl.ds(c * S, S)])

    return sc_kernel


def _combine_body(g_ref, e_ref, go_ref, eo_ref):
    go_ref[...] = g_ref[0, :] + g_ref[1, :]
    eo_ref[...] = e_ref[0, :] + e_ref[1, :]


def kernel(nxyz, bonds, bond_len, num_bonds):
    n_atoms = nxyz.shape[0]
    n_bonds = bonds.shape[0]
    n_mol = num_bonds.shape[0]

    G = ((n_atoms + 2 + 127) // 128) * 128
    S = ((n_mol + 1 + 15) // 16) * 16
    rps = G // _NS

    chunks_total = -(-n_bonds // _C)
    n_chunks = -(-chunks_total // _NW)
    n_chunks += n_chunks % 2
    n_pad = n_chunks * _NW * _C

    tbl = jnp.concatenate(
        [nxyz.reshape(-1), jnp.zeros(((G - n_atoms) * 4,), jnp.float32)])

    pad = n_pad - n_bonds
    src = jnp.concatenate(
        [bonds[:, 0], jnp.full((pad,), n_atoms, jnp.int32)])
    dst = jnp.concatenate(
        [bonds[:, 1], jnp.full((pad,), n_atoms + 1, jnp.int32)])
    starts = jnp.cumsum(num_bonds)[:-1]
    mark = jnp.zeros((n_bonds,), jnp.int32).at[starts].add(1)
    seg = jnp.concatenate([
        jnp.cumsum(mark, dtype=jnp.int32),
        jnp.full((pad,), n_mol, jnp.int32),
    ])
    r0 = jnp.concatenate([bond_len[:, 0], jnp.zeros((pad,), jnp.float32)])
    zeros = jnp.zeros((rps,), jnp.float32)

    sc_kernel = _make_sc_kernel(n_chunks, G, S, rps)
    gpart, epart = sc_kernel(
        tbl,
        src.reshape(_NW, -1, _C), dst.reshape(_NW, -1, _C),
        seg.reshape(_NW, -1, _C), r0, zeros)

    gsum, esum = pl.pallas_call(
        _combine_body,
        out_shape=[
            jax.ShapeDtypeStruct((3 * G,), jnp.float32),
            jax.ShapeDtypeStruct((S,), jnp.float32),
        ],
    )(gpart.reshape(_NC, 3 * G), epart.reshape(_NC, S))

    g3 = gsum.reshape(3, G)
    energy_grad = jnp.stack(
        [g3[0, :n_atoms], g3[1, :n_atoms], g3[2, :n_atoms]], axis=1)
    E = esum[:n_mol].reshape(n_mol, 1)
    return E, energy_grad
